# baseline (speedup 1.0000x reference)
import functools

import jax
import jax.numpy as jnp
from jax import lax
from jax.experimental import pallas as pl
from jax.experimental.pallas import tpu as pltpu
from jax.experimental.pallas import tpu_sc as plsc

VOCAB = 1000000
EMB = 64
HID = 128
K = 50
B = 16384

NC = 2
NS = 16
L = 16
NW = NC * NS
RPW = B // NW
G = 4
NG = RPW // G
GI = G * K
NV = -(-GI // L)
GIP = NV * L
CHUNKS = EMB // L

RPS = VOCAB // NS
CCH = 250
NCH = RPS // CCH
CU = 5


def _weighted_accum(rows_ref, wregs, acc_ref, i, r):
    accs = [jnp.zeros((L,), jnp.float32) for _ in range(CHUNKS)]
    for k in range(K):
        p = r * K + k
        wk = jnp.full((L,), wregs[p // L][p % L], jnp.float32)
        for c2 in range(EMB // 32):
            packed = rows_ref[p, pl.ds(c2 * 32, 32)]
            lo, hi = plsc.unpack(packed, format=plsc.PackFormat.INTERLEAVED)
            accs[2 * c2] = accs[2 * c2] + wk * lo
            accs[2 * c2 + 1] = accs[2 * c2 + 1] + wk * hi
    ibase = pl.multiple_of(i * EMB, EMB)
    for c in range(CHUNKS):
        acc_ref[pl.ds(ibase + c * L, L)] = accs[c]


def _pool_body(x_hbm, w_hbm, tblf_hbm, out_hbm, tblb_hbm,
               i0, i1, w0, w1, r0, r1, acc_v, ci0, ci1, co0, co1,
               si0, si1, sw0, sw1, sg0, sg1, sci0, sci1, sco0, sco1):
    idx_v = (i0, i1)
    w_v = (w0, w1)
    rows_v = (r0, r1)
    cin_v = (ci0, ci1)
    cout_v = (co0, co1)
    sems_i = (si0, si1)
    sems_w = (sw0, sw1)
    sems_g = (sg0, sg1)
    sems_ci = (sci0, sci1)
    sems_co = (sco0, sco1)
    sid = lax.axis_index("s")
    wid = sid * NC + lax.axis_index("c")
    base = wid * RPW

    rbase = sid * RPS

    def cin_src(c):
        return tblf_hbm.at[pl.ds((rbase + c * CCH) * EMB, CCH * EMB)]

    def cout_dst(c):
        return tblb_hbm.at[pl.ds(rbase + c * CCH, CCH)]

    for c in (0, 1):
        pltpu.async_copy(cin_src(c), cin_v[c], sems_ci[c])

    def cstep(h, carry):
        for buf in range(2):
            c = h * 2 + buf
            pltpu.make_async_copy(cin_src(c), cin_v[buf], sems_ci[buf]).wait()

            @pl.when(c >= 2)
            def _():
                pltpu.make_async_copy(cout_v[buf], cout_dst(c - 2),
                                      sems_co[buf]).wait()

            def pbody(v, carry2):
                rb = pl.multiple_of(v * (CU * EMB), EMB)
                for u in range(CU):
                    off = rb + u * EMB
                    row = v * CU + u
                    for c2 in range(EMB // 32):
                        a = cin_v[buf][pl.ds(off + c2 * 32, L)]
                        bb = cin_v[buf][pl.ds(off + c2 * 32 + L, L)]
                        cout_v[buf][row, pl.ds(c2 * 32, 32)] = plsc.pack(
                            a, bb, format=plsc.PackFormat.INTERLEAVED)
                return carry2

            lax.fori_loop(0, CCH // CU, pbody, 0)
            pltpu.async_copy(cout_v[buf], cout_dst(c), sems_co[buf])

            @pl.when(c + 2 < NCH)
            def _():
                pltpu.async_copy(cin_src(c + 2), cin_v[buf], sems_ci[buf])
        return carry

    lax.fori_loop(0, NCH // 2, cstep, 0)
    for buf in range(2):
        pltpu.make_async_copy(cout_v[buf], cout_dst(NCH - 2 + buf),
                              sems_co[buf]).wait()
    plsc.subcore_barrier()

    def idx_src(g):
        return x_hbm.at[pl.ds((base + g * G) * K, GI)]

    def w_src(g):
        return w_hbm.at[pl.ds((base + g * G) * K, GI)]

    def load_idx_vreg(buf, s):
        t = idx_v[buf][pl.ds(s * L, L)]
        if (s + 1) * L > GI:
            t = jnp.where(lax.iota(jnp.int32, L) < GI - s * L, t, 0)
        return t

    def start_gather(buf):
        for s in range(NV):
            pltpu.async_copy(
                tblb_hbm.at[load_idx_vreg(buf, s)],
                rows_v[buf].at[pl.ds(s * L, L)], sems_g[buf])

    def wait_gather(buf):
        for s in range(NV):
            pltpu.make_async_copy(
                tblb_hbm.at[load_idx_vreg(buf, s)],
                rows_v[buf].at[pl.ds(s * L, L)], sems_g[buf]).wait()

    for g in (0, 1):
        pltpu.async_copy(idx_src(g), idx_v[g].at[pl.ds(0, GI)], sems_i[g])
        pltpu.async_copy(w_src(g), w_v[g].at[pl.ds(0, GI)], sems_w[g])
    pltpu.make_async_copy(idx_src(0), idx_v[0].at[pl.ds(0, GI)],
                          sems_i[0]).wait()
    start_gather(0)

    def step(h, carry):
        for buf in range(2):
            g = h * 2 + buf
            nbuf = 1 - buf
            wait_gather(buf)

            @pl.when(g + 1 < NG)
            def _():
                pltpu.make_async_copy(
                    idx_src(g + 1), idx_v[nbuf].at[pl.ds(0, GI)],
                    sems_i[nbuf]).wait()
                start_gather(nbuf)

            @pl.when(g + 2 < NG)
            def _():
                pltpu.async_copy(idx_src(g + 2),
                                 idx_v[buf].at[pl.ds(0, GI)], sems_i[buf])

            pltpu.make_async_copy(w_src(g), w_v[buf].at[pl.ds(0, GI)],
                                  sems_w[buf]).wait()
            wregs = [w_v[buf][pl.ds(c * L, L)] for c in range(NV)]
            for r in range(G):
                _weighted_accum(rows_v[buf], wregs, acc_v, g * G + r, r)

            @pl.when(g + 2 < NG)
            def _():
                pltpu.async_copy(w_src(g + 2),
                                 w_v[buf].at[pl.ds(0, GI)], sems_w[buf])
        return carry

    lax.fori_loop(0, NG // 2, step, 0)
    pltpu.sync_copy(acc_v, out_hbm.at[pl.ds(base * EMB, RPW * EMB)])


@functools.lru_cache(maxsize=1)
def _get_pool():
    return functools.partial(
        pl.kernel,
        out_type=(jax.ShapeDtypeStruct((B * EMB,), jnp.float32),
                  jax.ShapeDtypeStruct((VOCAB, EMB), jnp.bfloat16)),
        mesh=plsc.VectorSubcoreMesh(core_axis_name="c", subcore_axis_name="s",
                                    num_cores=NC, num_subcores=NS),
        compiler_params=pltpu.CompilerParams(use_tc_tiling_on_sc=False,
                                             needs_layout_passes=False),
        scratch_types=[
            pltpu.VMEM((GIP,), jnp.int32),
            pltpu.VMEM((GIP,), jnp.int32),
            pltpu.VMEM((GIP,), jnp.float32),
            pltpu.VMEM((GIP,), jnp.float32),
            pltpu.VMEM((GIP, EMB), jnp.bfloat16),
            pltpu.VMEM((GIP, EMB), jnp.bfloat16),
            pltpu.VMEM((RPW * EMB,), jnp.float32),
            pltpu.VMEM((CCH * EMB,), jnp.float32),
            pltpu.VMEM((CCH * EMB,), jnp.float32),
            pltpu.VMEM((CCH, EMB), jnp.bfloat16),
            pltpu.VMEM((CCH, EMB), jnp.bfloat16),
            pltpu.SemaphoreType.DMA,
            pltpu.SemaphoreType.DMA,
            pltpu.SemaphoreType.DMA,
            pltpu.SemaphoreType.DMA,
            pltpu.SemaphoreType.DMA,
            pltpu.SemaphoreType.DMA,
            pltpu.SemaphoreType.DMA,
            pltpu.SemaphoreType.DMA,
            pltpu.SemaphoreType.DMA,
            pltpu.SemaphoreType.DMA,
        ],
    )(_pool_body)


def _mlp_body(s_ref, w1_ref, b1_ref, w2_ref, b2_ref, o_ref):
    s = s_ref[...] * (1.0 / K)
    h = jnp.dot(s, w1_ref[...], preferred_element_type=jnp.float32)
    h = jnp.maximum(h + b1_ref[...], 0.0)
    o_ref[...] = jnp.dot(h, w2_ref[...],
                         preferred_element_type=jnp.float32) + b2_ref[...]


_MLP_BLK = 2048

_mlp = pl.pallas_call(
    _mlp_body,
    grid=(B // _MLP_BLK,),
    in_specs=[
        pl.BlockSpec((_MLP_BLK, EMB), lambda i: (i, 0)),
        pl.BlockSpec((EMB, HID), lambda i: (0, 0)),
        pl.BlockSpec((1, HID), lambda i: (0, 0)),
        pl.BlockSpec((HID, 2), lambda i: (0, 0)),
        pl.BlockSpec((1, 2), lambda i: (0, 0)),
    ],
    out_specs=pl.BlockSpec((_MLP_BLK, 2), lambda i: (i, 0)),
    out_shape=jax.ShapeDtypeStruct((B, 2), jnp.float32),
)


def kernel(x, logits, emb_table, W1, b1, W2, b2):
    xi = x.astype(jnp.int32).reshape(B * K)
    w = logits.reshape(B * K)
    tblf = emb_table.reshape(VOCAB * EMB)
    s, _ = _get_pool()(xi, w, tblf)
    s = s.reshape(B, EMB)
    return _mlp(s, W1, b1.reshape(1, HID), W2, b2.reshape(1, 2))


# speedup vs baseline: 1.2845x; 1.2845x over previous
import functools

import jax
import jax.numpy as jnp
import numpy as np
from jax import lax
from jax.experimental import pallas as pl
from jax.experimental.pallas import tpu as pltpu
from jax.experimental.pallas import tpu_sc as plsc

VOCAB = 1000000
EMB = 64
HID = 128
K = 50
B = 16384

NC = 2
NS = 16
L = 16
NW = NC * NS
RPW = B // NW
G = 4
NG = RPW // G
GI = G * K
NV = -(-GI // L)
GIP = NV * L
CHUNKS = EMB // L

_PERM = np.concatenate([
    np.arange(0, 32, 2), np.arange(1, 32, 2),
    np.arange(32, 64, 2), np.arange(33, 64, 2),
])


def _weighted_accum(rows_ref, wregs, acc_ref, i, r):
    accs = [jnp.zeros((L,), jnp.float32) for _ in range(CHUNKS)]
    for k in range(K):
        p = r * K + k
        wk = jnp.full((L,), wregs[p // L][p % L], jnp.float32)
        for c2 in range(EMB // 32):
            packed = rows_ref[p, pl.ds(c2 * 32, 32)]
            lo, hi = plsc.unpack(packed, format=plsc.PackFormat.INTERLEAVED)
            accs[2 * c2] = accs[2 * c2] + wk * lo
            accs[2 * c2 + 1] = accs[2 * c2 + 1] + wk * hi
    ibase = pl.multiple_of(i * EMB, EMB)
    for c in range(CHUNKS):
        acc_ref[pl.ds(ibase + c * L, L)] = accs[c]


def _pool_body(x_hbm, w_hbm, table_hbm, out_hbm,
               i0, i1, w0, w1, r0, r1, acc_v,
               si0, si1, sw0, sw1, sg0, sg1):
    idx_v = (i0, i1)
    w_v = (w0, w1)
    rows_v = (r0, r1)
    sems_i = (si0, si1)
    sems_w = (sw0, sw1)
    sems_g = (sg0, sg1)
    wid = lax.axis_index("s") * NC + lax.axis_index("c")
    base = wid * RPW

    def idx_src(g):
        return x_hbm.at[pl.ds((base + g * G) * K, GI)]

    def w_src(g):
        return w_hbm.at[pl.ds((base + g * G) * K, GI)]

    def load_idx_vreg(buf, s):
        t = idx_v[buf][pl.ds(s * L, L)]
        if (s + 1) * L > GI:
            t = jnp.where(lax.iota(jnp.int32, L) < GI - s * L, t, 0)
        return t

    def start_gather(buf):
        for s in range(NV):
            pltpu.async_copy(
                table_hbm.at[load_idx_vreg(buf, s)],
                rows_v[buf].at[pl.ds(s * L, L)], sems_g[buf])

    def wait_gather(buf):
        for s in range(NV):
            pltpu.make_async_copy(
                table_hbm.at[load_idx_vreg(buf, s)],
                rows_v[buf].at[pl.ds(s * L, L)], sems_g[buf]).wait()

    for g in (0, 1):
        pltpu.async_copy(idx_src(g), idx_v[g].at[pl.ds(0, GI)], sems_i[g])
        pltpu.async_copy(w_src(g), w_v[g].at[pl.ds(0, GI)], sems_w[g])
    pltpu.make_async_copy(idx_src(0), idx_v[0].at[pl.ds(0, GI)],
                          sems_i[0]).wait()
    start_gather(0)

    def step(h, carry):
        for buf in range(2):
            g = h * 2 + buf
            nbuf = 1 - buf
            @pl.when(g + 1 < NG)
            def _():
                pltpu.make_async_copy(
                    idx_src(g + 1), idx_v[nbuf].at[pl.ds(0, GI)],
                    sems_i[nbuf]).wait()
                start_gather(nbuf)

            wait_gather(buf)

            @pl.when(g + 2 < NG)
            def _():
                pltpu.async_copy(idx_src(g + 2),
                                 idx_v[buf].at[pl.ds(0, GI)], sems_i[buf])

            pltpu.make_async_copy(w_src(g), w_v[buf].at[pl.ds(0, GI)],
                                  sems_w[buf]).wait()
            wregs = [w_v[buf][pl.ds(c * L, L)] for c in range(NV)]
            for r in range(G):
                _weighted_accum(rows_v[buf], wregs, acc_v, g * G + r, r)

            @pl.when(g + 2 < NG)
            def _():
                pltpu.async_copy(w_src(g + 2),
                                 w_v[buf].at[pl.ds(0, GI)], sems_w[buf])
        return carry

    lax.fori_loop(0, NG // 2, step, 0)
    pltpu.sync_copy(acc_v, out_hbm.at[pl.ds(base * EMB, RPW * EMB)])


@functools.lru_cache(maxsize=1)
def _get_pool():
    return functools.partial(
        pl.kernel,
        out_type=jax.ShapeDtypeStruct((B * EMB,), jnp.float32),
        mesh=plsc.VectorSubcoreMesh(core_axis_name="c", subcore_axis_name="s",
                                    num_cores=NC, num_subcores=NS),
        compiler_params=pltpu.CompilerParams(use_tc_tiling_on_sc=False,
                                             needs_layout_passes=False),
        scratch_types=[
            pltpu.VMEM((GIP,), jnp.int32),
            pltpu.VMEM((GIP,), jnp.int32),
            pltpu.VMEM((GIP,), jnp.float32),
            pltpu.VMEM((GIP,), jnp.float32),
            pltpu.VMEM((GIP, EMB), jnp.bfloat16),
            pltpu.VMEM((GIP, EMB), jnp.bfloat16),
            pltpu.VMEM((RPW * EMB,), jnp.float32),
            pltpu.SemaphoreType.DMA,
            pltpu.SemaphoreType.DMA,
            pltpu.SemaphoreType.DMA,
            pltpu.SemaphoreType.DMA,
            pltpu.SemaphoreType.DMA,
            pltpu.SemaphoreType.DMA,
        ],
    )(_pool_body)


def _mlp_body(s_ref, w1_ref, b1_ref, w2_ref, b2_ref, o_ref):
    s = s_ref[...] * (1.0 / K)
    h = jnp.dot(s, w1_ref[...], preferred_element_type=jnp.float32)
    h = jnp.maximum(h + b1_ref[...], 0.0)
    o_ref[...] = jnp.dot(h, w2_ref[...],
                         preferred_element_type=jnp.float32) + b2_ref[...]


_MLP_BLK = 2048

_mlp = pl.pallas_call(
    _mlp_body,
    grid=(B // _MLP_BLK,),
    in_specs=[
        pl.BlockSpec((_MLP_BLK, EMB), lambda i: (i, 0)),
        pl.BlockSpec((EMB, HID), lambda i: (0, 0)),
        pl.BlockSpec((1, HID), lambda i: (0, 0)),
        pl.BlockSpec((HID, 2), lambda i: (0, 0)),
        pl.BlockSpec((1, 2), lambda i: (0, 0)),
    ],
    out_specs=pl.BlockSpec((_MLP_BLK, 2), lambda i: (i, 0)),
    out_shape=jax.ShapeDtypeStruct((B, 2), jnp.float32),
)


def kernel(x, logits, emb_table, W1, b1, W2, b2):
    xi = x.astype(jnp.int32).reshape(B * K)
    w = logits.reshape(B * K)
    tbl = emb_table.astype(jnp.bfloat16)
    s = _get_pool()(xi, w, tbl).reshape(B, EMB)
    return _mlp(s, W1[_PERM], b1.reshape(1, HID), W2, b2.reshape(1, 2))
